# Initial kernel scaffold; baseline (speedup 1.0000x reference)
import jax
import jax.numpy as jnp
from jax.experimental import pallas as pl


def kernel(x, embed):
    raise NotImplementedError("write your pallas kernel here")


# speedup vs baseline: 1.2627x; 1.2627x over previous
import functools

import jax
import jax.numpy as jnp
from jax import lax
from jax.experimental import pallas as pl
from jax.experimental.pallas import tpu as pltpu
from jax.experimental.pallas import tpu_sc as plsc

N = 36864
D = 64
K = 1024


TB = 1024
G = N // TB


def _argmin_body(x_ref, e_ref, out_ref):
    x = x_ref[...]
    e = e_ref[...]
    xe = lax.dot_general(
        x, e, (((1,), (1,)), ((), ())),
        preferred_element_type=jnp.float32,
    )
    s1 = jnp.sum(x * x, axis=1, keepdims=True)
    s2 = jnp.sum(e * e, axis=1)[None, :]
    dist = -(s1 - 2.0 * xe + s2)
    m = jnp.max(dist, axis=1, keepdims=True)
    ids = lax.broadcasted_iota(jnp.int32, dist.shape, 1)
    cand = jnp.where(dist == m, ids, jnp.int32(1 << 30))
    out_ref[0, 0, :] = jnp.min(cand, axis=1)


def _argmin_indices(flat, embed):
    return pl.pallas_call(
        _argmin_body,
        grid=(G,),
        in_specs=[
            pl.BlockSpec((TB, D), lambda i: (i, 0)),
            pl.BlockSpec((K, D), lambda i: (0, 0)),
        ],
        out_specs=pl.BlockSpec((1, 1, TB), lambda i: (i, 0, 0)),
        out_shape=jax.ShapeDtypeStruct((G, 1, TB), jnp.int32),
    )(flat, embed)


NC = 2
NS = 16
NW = NC * NS
CHUNK = 128
ROWS_PER_W = N // NW
CHUNKS_PER_W = ROWS_PER_W // CHUNK

def _gather_body(table_hbm, idx_hbm, out_hbm, idx_v, rows_v, sem):
    wid = lax.axis_index("s") * NC + lax.axis_index("c")
    pltpu.sync_copy(idx_hbm.at[wid], idx_v)
    copies = []
    for j in range(CHUNKS_PER_W):
        copies.append(pltpu.async_copy(
            table_hbm.at[idx_v.at[j]],
            rows_v.at[pl.ds(j * CHUNK, CHUNK)],
            sem,
        ))
    for c in copies:
        c.wait()
    pltpu.sync_copy(rows_v, out_hbm.at[pl.ds(wid * ROWS_PER_W, ROWS_PER_W)])


@functools.cache
def _sc_gather_fn():
    mesh = plsc.VectorSubcoreMesh(
        core_axis_name="c", subcore_axis_name="s",
        num_cores=NC, num_subcores=NS)
    return pl.kernel(
        _gather_body,
        out_type=jax.ShapeDtypeStruct((N, D), jnp.float32),
        mesh=mesh,
        scratch_types=[
            pltpu.VMEM((CHUNKS_PER_W, CHUNK), jnp.int32),
            pltpu.VMEM((ROWS_PER_W, D), jnp.float32),
            pltpu.SemaphoreType.DMA,
        ],
        compiler_params=pltpu.CompilerParams(use_tc_tiling_on_sc=False),
    )


def kernel(x, embed):
    shape = x.shape
    flat = x.reshape(-1, shape[-1])
    idx = _argmin_indices(flat, embed).reshape(NW, CHUNKS_PER_W, CHUNK)
    out = _sc_gather_fn()(embed, idx)
    return out.reshape(shape)


# speedup vs baseline: 1.3040x; 1.0327x over previous
import functools

import jax
import jax.numpy as jnp
from jax import lax
from jax.experimental import pallas as pl
from jax.experimental.pallas import tpu as pltpu
from jax.experimental.pallas import tpu_sc as plsc

N = 36864
D = 64
K = 1024


TB = 1024
G = N // TB


KC = 128
NKC = K // KC


def _argmin_body(x_ref, e_ref, out_ref):
    x = x_ref[...]
    s1 = jnp.sum(x * x, axis=1, keepdims=True)
    lane = lax.broadcasted_iota(jnp.int32, (TB, KC), 1)
    best_v = None
    for kc in range(NKC):
        e = e_ref[pl.ds(kc * KC, KC), :]
        xe = lax.dot_general(
            x, e, (((1,), (1,)), ((), ())),
            preferred_element_type=jnp.float32,
        )
        s2 = jnp.sum(e * e, axis=1)[None, :]
        dist = -(s1 - 2.0 * xe + s2)
        idc = lane + (kc * KC)
        if best_v is None:
            best_v, best_i = dist, idc
        else:
            take = dist > best_v
            best_v = jnp.where(take, dist, best_v)
            best_i = jnp.where(take, idc, best_i)
    m = jnp.max(best_v, axis=1, keepdims=True)
    cand = jnp.where(best_v == m, best_i, jnp.int32(1 << 30))
    out_ref[0, 0, :] = jnp.min(cand, axis=1)


def _argmin_indices(flat, embed):
    return pl.pallas_call(
        _argmin_body,
        grid=(G,),
        in_specs=[
            pl.BlockSpec((TB, D), lambda i: (i, 0)),
            pl.BlockSpec((K, D), lambda i: (0, 0)),
        ],
        out_specs=pl.BlockSpec((1, 1, TB), lambda i: (i, 0, 0)),
        out_shape=jax.ShapeDtypeStruct((G, 1, TB), jnp.int32),
    )(flat, embed)


NC = 2
NS = 16
NW = NC * NS
CHUNK = 128
ROWS_PER_W = N // NW
CHUNKS_PER_W = ROWS_PER_W // CHUNK

def _gather_body(table_hbm, idx_hbm, out_hbm, idx_v, rows_v, sem):
    wid = lax.axis_index("s") * NC + lax.axis_index("c")
    pltpu.sync_copy(idx_hbm.at[wid], idx_v)
    copies = []
    for j in range(CHUNKS_PER_W):
        copies.append(pltpu.async_copy(
            table_hbm.at[idx_v.at[j]],
            rows_v.at[pl.ds(j * CHUNK, CHUNK)],
            sem,
        ))
    for c in copies:
        c.wait()
    pltpu.sync_copy(rows_v, out_hbm.at[pl.ds(wid * ROWS_PER_W, ROWS_PER_W)])


@functools.cache
def _sc_gather_fn():
    mesh = plsc.VectorSubcoreMesh(
        core_axis_name="c", subcore_axis_name="s",
        num_cores=NC, num_subcores=NS)
    return pl.kernel(
        _gather_body,
        out_type=jax.ShapeDtypeStruct((N, D), jnp.float32),
        mesh=mesh,
        scratch_types=[
            pltpu.VMEM((CHUNKS_PER_W, CHUNK), jnp.int32),
            pltpu.VMEM((ROWS_PER_W, D), jnp.float32),
            pltpu.SemaphoreType.DMA,
        ],
        compiler_params=pltpu.CompilerParams(use_tc_tiling_on_sc=False),
    )


def kernel(x, embed):
    shape = x.shape
    flat = x.reshape(-1, shape[-1])
    idx = _argmin_indices(flat, embed).reshape(NW, CHUNKS_PER_W, CHUNK)
    out = _sc_gather_fn()(embed, idx)
    return out.reshape(shape)
